# baseline (speedup 1.0000x reference)
import functools

import jax
import jax.numpy as jnp
from jax import lax
from jax.experimental import pallas as pl
from jax.experimental.pallas import tpu as pltpu
from jax.experimental.pallas import tpu_sc as plsc

_D = 32
_B = 16384 * 26
_NW = 32
_BPW = _B // _NW
_C = 64
_NCHUNK = _BPW // _C
_NBUF = 4
_PRE = 3
_QROWS = 250000


def _build():
    mesh = plsc.VectorSubcoreMesh(core_axis_name="c", subcore_axis_name="s")

    @functools.partial(
        pl.kernel,
        mesh=mesh,
        out_type=jax.ShapeDtypeStruct((_B, _D), jnp.float32),
        scratch_types=[
            pltpu.VMEM((_BPW,), jnp.int32),
            pltpu.VMEM((_NBUF, _C), jnp.int32),
            pltpu.VMEM((_NBUF, _C, 128), jnp.float32),
            pltpu.VMEM((_NBUF, _C, _D), jnp.float32),
            pltpu.SemaphoreType.DMA((_NBUF,)),
            pltpu.SemaphoreType.DMA((_NBUF,)),
        ],
    )
    def k(idx_hbm, tbl_hbm, out_hbm, idx_v, qid_v, buf_v, stage_v,
          g_sem, s_sem):
        wid = lax.axis_index("s") * 2 + lax.axis_index("c")
        base = wid * _BPW
        pltpu.sync_copy(idx_hbm.at[pl.ds(base, _BPW)], idx_v)

        def gather(c, b):
            for j in range(_C // 16):
                v = idx_v[pl.ds(c * _C + j * 16, 16)]
                qid_v[b, pl.ds(j * 16, 16)] = v >> 2
            pltpu.async_copy(
                tbl_hbm.at[qid_v.at[b]], buf_v.at[b], g_sem.at[b])

        def wait_gather(b):
            pltpu.make_async_copy(
                tbl_hbm.at[pl.ds(0, _C)], buf_v.at[b], g_sem.at[b]).wait()

        def store(c, b):
            pltpu.async_copy(
                stage_v.at[b], out_hbm.at[pl.ds(base + c * _C, _C)],
                s_sem.at[b])

        def wait_store(b):
            pltpu.make_async_copy(
                stage_v.at[b], out_hbm.at[pl.ds(0, _C)], s_sem.at[b]).wait()

        def extract(c, b):
            @pl.loop(0, _C // 16)
            def _grp(j):
                ivs = idx_v[pl.ds(c * _C + j * 16, 16)]
                offs = (ivs & 3) * _D
                for q in range(16):
                    off = offs[q]
                    r = j * 16 + q
                    stage_v[b, r, pl.ds(0, 16)] = buf_v[b, r, pl.ds(off, 16)]
                    stage_v[b, r, pl.ds(16, 16)] = (
                        buf_v[b, r, pl.ds(off + 16, 16)])

        for c in range(_PRE):
            gather(c, c % _NBUF)
        for c in range(_NBUF):
            b = c % _NBUF
            wait_gather(b)
            extract(c, b)
            store(c, b)
            if c + _PRE < _NCHUNK:
                gather(c + _PRE, (c + _PRE) % _NBUF)

        @pl.loop(_NBUF, _NCHUNK, step=_NBUF)
        def _main(g):
            for b in range(_NBUF):
                c = g + b
                wait_gather(b)
                wait_store(b)
                extract(c, b)
                store(c, b)

                @pl.when(c + _PRE < _NCHUNK)
                def _():
                    gather(c + _PRE, (b + _PRE) % _NBUF)

        for b in range(_NBUF):
            wait_store(b)

    return k


_gather_call = _build()


@jax.jit
def kernel(x, table):
    idx = x.reshape(-1)
    tbl = table.reshape(_QROWS, 128)
    out = _gather_call(idx, tbl)
    return out.reshape(x.shape + (table.shape[1],))


# speedup vs baseline: 1.2415x; 1.2415x over previous
import functools

import jax
import jax.numpy as jnp
from jax import lax
from jax.experimental import pallas as pl
from jax.experimental.pallas import tpu as pltpu
from jax.experimental.pallas import tpu_sc as plsc

_D = 32
_B = 16384 * 26
_NW = 32
_BPW = _B // _NW
_C = 256
_NCHUNK = _BPW // _C
_NBUF = 8
_PRE = 6


def _build():
    mesh = plsc.VectorSubcoreMesh(core_axis_name="c", subcore_axis_name="s")

    @functools.partial(
        pl.kernel,
        mesh=mesh,
        compiler_params=pltpu.CompilerParams(use_tc_tiling_on_sc=False),
        out_type=jax.ShapeDtypeStruct((_B, _D), jnp.float32),
        scratch_types=[
            pltpu.VMEM((_BPW,), jnp.int32),
            pltpu.VMEM((_NBUF, _C, _D), jnp.float32),
            pltpu.SemaphoreType.DMA((_NBUF,)),
            pltpu.SemaphoreType.DMA((_NBUF,)),
        ],
    )
    def k(idx_hbm, table_hbm, out_hbm, idx_v, rows_v, g_sem, s_sem):
        wid = lax.axis_index("s") * 2 + lax.axis_index("c")
        base = wid * _BPW
        pltpu.sync_copy(idx_hbm.at[pl.ds(base, _BPW)], idx_v)

        def gather(c, buf):
            return pltpu.async_copy(
                table_hbm.at[idx_v.at[pl.ds(c * _C, _C)]],
                rows_v.at[buf], g_sem.at[buf])

        def store(c, buf):
            return pltpu.async_copy(
                rows_v.at[buf], out_hbm.at[pl.ds(base + c * _C, _C)],
                s_sem.at[buf])

        g = [None] * _NCHUNK
        s = [None] * _NCHUNK
        for c in range(_PRE):
            g[c] = gather(c, c % _NBUF)
        for c in range(_NCHUNK):
            g[c].wait()
            s[c] = store(c, c % _NBUF)
            nxt = c + _PRE
            if nxt < _NCHUNK:
                old = nxt - _NBUF
                if old >= 0:
                    s[old].wait()
                g[nxt] = gather(nxt, nxt % _NBUF)
        for c in range(max(0, _NCHUNK - _NBUF), _NCHUNK):
            s[c].wait()

    return k


_gather_call = _build()


@jax.jit
def kernel(x, table):
    idx = x.reshape(-1)
    out = _gather_call(idx, table)
    return out.reshape(x.shape + (table.shape[1],))
